# baseline (speedup 1.0000x reference)
import functools

import jax
import jax.numpy as jnp
from jax import lax
from jax.experimental import pallas as pl
from jax.experimental.pallas import tpu as pltpu
from jax.experimental.pallas import tpu_sc as plsc

_NBUF = 4
_NL = 128
_UNROLL = 8


@functools.cache
def _make_gather(N, T, D):
    info = plsc.get_sparse_core_info()
    NC, NS = info.num_cores, info.num_subcores
    NW = NC * NS
    G = D // 8
    TILE = 8 * _NL
    CH = G * TILE
    NB = N // _NL
    assert NB % NW == 0
    nb_w = NB // NW
    n_chunks = T * nb_w
    assert n_chunks % _NBUF == 0
    n_groups = n_chunks // _NBUF
    assert n_groups >= 2

    mesh = plsc.VectorSubcoreMesh(core_axis_name="c", subcore_axis_name="s")

    @functools.partial(
        pl.kernel,
        mesh=mesh,
        out_type=jax.ShapeDtypeStruct((T, G, NB, TILE), jnp.float32),
        scratch_types=(
            [pltpu.VMEM((_NBUF, _NL), jnp.int32),
             pltpu.VMEM((_NBUF, _NL, D), jnp.float32),
             pltpu.VMEM((_NBUF * CH,), jnp.float32)]
            + [pltpu.SemaphoreType.DMA] * (3 * _NBUF)
        ),
        compiler_params=pltpu.CompilerParams(
            use_tc_tiling_on_sc=False, needs_layout_passes=False,
            disable_bounds_checks=True),
    )
    def k(table_hbm, idx_hbm, out_hbm, idx_v, rows_v, tbuf, *sems):
        sem_idx = sems[:_NBUF]
        sem_g = sems[_NBUF:2 * _NBUF]
        sem_out = sems[2 * _NBUF:]
        wid = lax.axis_index("s") * NC + lax.axis_index("c")
        nb0 = wid * nb_w

        iota = lax.iota(jnp.int32, 16)

        def tnb(c):
            return c // nb_w, nb0 + c % nb_w

        def idx_copy(c, b):
            t, nb = tnb(c)
            return pltpu.make_async_copy(
                idx_hbm.at[t, pl.ds(nb * _NL, _NL)], idx_v.at[b], sem_idx[b])

        def gather_copy(b):
            return pltpu.make_async_copy(
                table_hbm.at[idx_v.at[b]], rows_v.at[b], sem_g[b])

        def out_copies(c, b):
            t, nb = tnb(c)
            return [pltpu.make_async_copy(
                tbuf.at[pl.ds(b * CH + g * TILE, TILE)],
                out_hbm.at[t, g, nb], sem_out[b])
                for g in range(G)]

        def transpose(b):
            i0 = b * CH + iota * _NL
            i1 = i0 + 16 * _NL

            def body(i, idxs):
                j0, j1 = idxs
                for u in range(_UNROLL):
                    r = i * _UNROLL + u
                    plsc.store_scatter(tbuf, [j0], rows_v[b, r, pl.ds(0, 16)])
                    plsc.store_scatter(tbuf, [j1], rows_v[b, r, pl.ds(16, 16)])
                    j0 = j0 + 1
                    j1 = j1 + 1
                return j0, j1

            lax.fori_loop(0, _NL // _UNROLL, body, (i0, i1))

        for b in range(_NBUF):
            idx_copy(b, b).start()

        for b in range(_NBUF):
            idx_copy(b, b).wait()
            gather_copy(b).start()
        for b in range(_NBUF):
            gather_copy(b).wait()
            transpose(b)
            for cp in out_copies(b, b):
                cp.start()
            idx_copy(_NBUF + b, b).start()

        def group(g, carry):
            j0 = g * _NBUF
            for b in range(_NBUF):
                for cp in out_copies(j0 - _NBUF + b, b):
                    cp.wait()
                idx_copy(j0 + b, b).wait()
                gather_copy(b).start()
            for b in range(_NBUF):
                gather_copy(b).wait()
                transpose(b)
                for cp in out_copies(j0 + b, b):
                    cp.start()
                idx_copy(j0 + _NBUF + b, b).start()
            return carry

        lax.fori_loop(1, n_groups - 1, group, 0)

        j0 = (n_groups - 1) * _NBUF
        for b in range(_NBUF):
            for cp in out_copies(j0 - _NBUF + b, b):
                cp.wait()
            idx_copy(j0 + b, b).wait()
            gather_copy(b).start()
        for b in range(_NBUF):
            gather_copy(b).wait()
            transpose(b)
            for cp in out_copies(j0 + b, b):
                cp.start()
        for b in range(_NBUF):
            for cp in out_copies(j0 + b, b):
                cp.wait()

    return k


def kernel(x, table):
    N, T = x.shape
    D = table.shape[1]
    xt = x.T.astype(jnp.int32)
    y = _make_gather(N, T, D)(table, xt)
    return (y.reshape(T, D // 8, N // 128, 8, 128)
            .transpose(2, 4, 0, 1, 3).reshape(N, T, D))


# speedup vs baseline: 1.1152x; 1.1152x over previous
import functools

import jax
import jax.numpy as jnp
from jax import lax
from jax.experimental import pallas as pl
from jax.experimental.pallas import tpu as pltpu
from jax.experimental.pallas import tpu_sc as plsc

_NBUF = 4
_CHUNK = 800


@functools.cache
def _make_gather(B, D):
    info = plsc.get_sparse_core_info()
    NC, NS = info.num_cores, info.num_subcores
    NW = NC * NS
    assert B % NW == 0
    per_w = B // NW
    C = _CHUNK
    assert per_w % (C * _NBUF) == 0
    n_groups = per_w // (C * _NBUF)
    assert n_groups >= 2

    mesh = plsc.VectorSubcoreMesh(core_axis_name="c", subcore_axis_name="s")

    @functools.partial(
        pl.kernel,
        mesh=mesh,
        out_type=jax.ShapeDtypeStruct((B, D), jnp.float32),
        scratch_types=(
            [pltpu.VMEM((_NBUF, C), jnp.int32),
             pltpu.VMEM((_NBUF, C, D), jnp.float32)]
            + [pltpu.SemaphoreType.DMA] * (3 * _NBUF)
        ),
        compiler_params=pltpu.CompilerParams(use_tc_tiling_on_sc=False),
    )
    def k(table_hbm, idx_hbm, out_hbm, idx_v, rows_v, *sems):
        sem_idx = sems[:_NBUF]
        sem_g = sems[_NBUF:2 * _NBUF]
        sem_out = sems[2 * _NBUF:]
        wid = lax.axis_index("s") * NC + lax.axis_index("c")
        base = wid * per_w

        def idx_copy(j, b):
            return pltpu.make_async_copy(
                idx_hbm.at[pl.ds(base + j * C, C)], idx_v.at[b], sem_idx[b])

        def gather_copy(b):
            return pltpu.make_async_copy(
                table_hbm.at[idx_v.at[b]], rows_v.at[b], sem_g[b])

        def out_copy(j, b):
            return pltpu.make_async_copy(
                rows_v.at[b], out_hbm.at[pl.ds(base + j * C, C)], sem_out[b])

        for b in range(_NBUF):
            idx_copy(b, b).start()

        for b in range(_NBUF):
            idx_copy(b, b).wait()
            gather_copy(b).start()
        for b in range(_NBUF):
            gather_copy(b).wait()
            out_copy(b, b).start()
            idx_copy(_NBUF + b, b).start()

        def group(g, carry):
            j0 = g * _NBUF
            for b in range(_NBUF):
                out_copy(j0 - _NBUF + b, b).wait()
                idx_copy(j0 + b, b).wait()
                gather_copy(b).start()
            for b in range(_NBUF):
                gather_copy(b).wait()
                out_copy(j0 + b, b).start()
                idx_copy(j0 + _NBUF + b, b).start()
            return carry

        lax.fori_loop(1, n_groups - 1, group, 0)

        j0 = (n_groups - 1) * _NBUF
        for b in range(_NBUF):
            out_copy(j0 - _NBUF + b, b).wait()
            idx_copy(j0 + b, b).wait()
            gather_copy(b).start()
        for b in range(_NBUF):
            gather_copy(b).wait()
            out_copy(j0 + b, b).start()
        for b in range(_NBUF):
            out_copy(j0 + b, b).wait()

    return k


def kernel(x, table):
    B = x.shape[0] * x.shape[1]
    D = table.shape[1]
    idx = x.reshape(B).astype(jnp.int32)
    out = _make_gather(B, D)(table, idx)
    return out.reshape(x.shape + (D,))
